# baseline (speedup 1.0000x reference)
import jax
import jax.numpy as jnp
import numpy as np
from jax.experimental import pallas as pl

G = 12
RN = G * 10
SUB = 32
SPG = G * SUB
ROWS = RN * SUB
STEPS = 3
PB = STEPS * SPG


def _ln(v, g, b):
    m = jnp.mean(v, axis=1, keepdims=True)
    var = jnp.mean((v - m) ** 2, axis=1, keepdims=True)
    return (v - m) / jnp.sqrt(var + 1e-5) * g + b


def _elu(v):
    return jnp.where(v > 0, v, jnp.exp(jnp.minimum(v, 0.0)) - 1.0)


def _gat_head(hw_h, a_s, a_d, CT):
    esr = jax.lax.dot_general(a_s, hw_h, (((1,), (1,)), ((), ())))
    edc = jax.lax.dot_general(hw_h, a_d, (((1,), (1,)), ((), ())))
    e = edc + esr
    e = jnp.maximum(e, 0.2 * e)
    cee = CT * jnp.exp(e)
    den = jnp.sum(cee, axis=1, keepdims=True)
    return jnp.dot(cee, hw_h) * (1.0 / (den + 1e-16))


def _actor_kernel(nf_ref, topo_ref, rt_ref, tf_ref,
                  segf, samegf, ind0f, rept,
                  we, be, gne, bne, wp, bp,
                  wg1, as1, ad1, bg1, g1, b1,
                  wg2, as2, ad2, bg2, g2, b2,
                  wr, br, gr, brb, wt, bt, gt, btb,
                  wf, bf, gf, bfb, wa1, ba1, wa2, ba2, wa3, ba3,
                  out_ref):
    nf = nf_ref[0]
    topo = topo_ref[0]

    h0 = jax.nn.relu(jnp.dot(nf, we[...]) + be[...])
    h0 = _ln(h0, gne[...], bne[...])
    ident = jnp.dot(h0, wp[...]) + bp[...]
    hw1 = jnp.dot(h0, wg1[...])

    mask = (topo != 0).astype(jnp.float32)
    rowsum = jnp.sum(mask, axis=1, keepdims=True)
    nnz = jnp.dot(segf[...], rowsum)
    padc = jax.lax.dot_general(100.0 - nnz, segf[...],
                               (((0,), (0,)), ((), ())))
    maskt = mask.T

    CTs = []
    for t in range(SUB):
        sl = slice(RN * t, RN * (t + 1))
        CTs.append(jnp.dot(rept[...], maskt[:, sl]) * samegf[...]
                   + ind0f[...] * padc[:, sl])

    x1_parts = []
    for t in range(SUB):
        sl = slice(RN * t, RN * (t + 1))
        outs = [_gat_head(hw1[sl, 64 * h:64 * (h + 1)],
                          as1[h:h + 1, :], ad1[h:h + 1, :], CTs[t])
                for h in range(4)]
        x1_parts.append(jnp.concatenate(outs, axis=1))
    x1 = jnp.concatenate(x1_parts, axis=0) + bg1[...]
    x1 = _elu(_ln(x1, g1[...], b1[...]))

    hw2 = jnp.dot(x1, wg2[...])
    x2 = jnp.concatenate(
        [_gat_head(hw2[RN * t:RN * (t + 1)], as2[...], ad2[...],
                   CTs[t]) for t in range(SUB)],
        axis=0) + bg2[...]
    x2 = _ln(x2, g2[...], b2[...])

    outg = _elu(x2 + ident)
    g = jnp.dot(segf[...], outg) * 0.1

    r = _ln(jax.nn.relu(jnp.dot(rt_ref[0], wr[...]) + br[...]),
            gr[...], brb[...])
    tt = _ln(jax.nn.relu(jnp.dot(tf_ref[0], wt[...]) + bt[...]),
             gt[...], btb[...])
    comb = jnp.concatenate([g, r, tt], axis=1)
    feat = _ln(jax.nn.relu(jnp.dot(comb, wf[...]) + bf[...]),
               gf[...], bfb[...])
    h1 = jax.nn.relu(jnp.dot(feat, wa1[...]) + ba1[...])
    h2 = jax.nn.relu(jnp.dot(h1, wa2[...]) + ba2[...])
    out_ref[0] = jnp.dot(h2, wa3[...]) + ba3[...]


def _full(a):
    a = jnp.asarray(a, jnp.float32)
    if a.ndim == 1:
        a = a.reshape(1, -1)
    return pl.BlockSpec(a.shape, lambda i: (0,) * a.ndim), a


@jax.jit
def kernel(x, params):
    p = params
    B = x.shape[0]
    xp = jnp.pad(x, ((0, PB - B), (0, 0)))
    topo = xp[:, :100].reshape(STEPS, ROWS, 10)
    nf = jnp.concatenate(
        [xp[:, 100:130].reshape(PB, 10, 3), xp[:, 245:255].reshape(PB, 10, 1)],
        axis=-1).reshape(STEPS, ROWS, 4)
    routing = xp[:, 130:140].reshape(STEPS, SPG, 10)
    traffic = xp[:, 240:245].reshape(STEPS, SPG, 5)

    s_of_row = np.arange(ROWS) // 10
    segf = (s_of_row[None, :] == np.arange(SPG)[:, None]).astype(np.float32)
    r1 = np.arange(RN)
    samegf = ((r1[:, None] // 10) == (r1[None, :] // 10)).astype(np.float32)
    ind0f = ((r1[:, None] % 10 == 0) & (r1[None, :] == r1[:, None])
             ).astype(np.float32)
    rept = ((r1[:, None] % 10) == np.arange(10)[None, :]).astype(np.float32)

    pnames = ['we', 'be', 'gne', 'bne', 'wp', 'bp',
              'wg1', 'as1', 'ad1', 'bg1', 'g1', 'b1',
              'wg2', 'as2', 'ad2', 'bg2', 'g2', 'b2',
              'wr', 'br', 'gr', 'brb', 'wt', 'bt', 'gt', 'btb',
              'wf', 'bf', 'gf', 'bfb', 'wa1', 'ba1', 'wa2', 'ba2',
              'wa3', 'ba3']
    cspecs, cvals = zip(*(_full(a) for a in
                          (segf, samegf, ind0f, rept)))
    pspecs, pvals = zip(*(_full(p[n]) for n in pnames))

    out = pl.pallas_call(
        _actor_kernel,
        grid=(STEPS,),
        in_specs=[pl.BlockSpec((1, ROWS, 4), lambda i: (i, 0, 0)),
                  pl.BlockSpec((1, ROWS, 10), lambda i: (i, 0, 0)),
                  pl.BlockSpec((1, SPG, 10), lambda i: (i, 0, 0)),
                  pl.BlockSpec((1, SPG, 5), lambda i: (i, 0, 0)),
                  *cspecs, *pspecs],
        out_specs=pl.BlockSpec((1, SPG, 10), lambda i: (i, 0, 0)),
        out_shape=jax.ShapeDtypeStruct((STEPS, SPG, 10), jnp.float32),
    )(nf, topo, routing, traffic, *cvals, *pvals)
    return out.reshape(PB, 10)[:B]


# speedup vs baseline: 1.0293x; 1.0293x over previous
import jax
import jax.numpy as jnp
import numpy as np
from jax.experimental import pallas as pl

G = 12
RN = G * 10
SUB = 16
SPG = G * SUB
ROWS = RN * SUB
STEPS = 6
PB = STEPS * SPG


def _ln(v, g, b):
    m = jnp.mean(v, axis=1, keepdims=True)
    var = jnp.mean((v - m) ** 2, axis=1, keepdims=True)
    return (v - m) / jnp.sqrt(var + 1e-5) * g + b


def _elu(v):
    return jnp.where(v > 0, v, jnp.exp(jnp.minimum(v, 0.0)) - 1.0)


def _gat_head(hw_h, a_s, edc, CT):
    esr = jax.lax.dot_general(a_s, hw_h, (((1,), (1,)), ((), ())))
    e = edc + esr
    e = jnp.maximum(e, 0.2 * e)
    cee = CT * jnp.exp(e)
    den = jnp.sum(cee, axis=1, keepdims=True)
    return jnp.dot(cee, hw_h) * (1.0 / (den + 1e-16))


def _actor_kernel(nf_ref, topo_ref, rt_ref, tf_ref,
                  segf, samegf, ind0f, rept,
                  we, be, gne, bne, wp, bp,
                  wg1, as1, ad1f, bg1, g1, b1,
                  wg2, as2, ad2f, bg2, g2, b2,
                  wr, br, gr, brb, wt, bt, gt, btb,
                  wf, bf, gf, bfb, wa1, ba1, wa2, ba2, wa3, ba3,
                  out_ref):
    nf = nf_ref[0]
    topo = topo_ref[0]

    h0 = jax.nn.relu(jnp.dot(nf, we[...]) + be[...])
    h0 = _ln(h0, gne[...], bne[...])
    ident = jnp.dot(h0, wp[...]) + bp[...]
    hw1 = jnp.dot(h0, wg1[...])
    edall1 = jnp.dot(hw1, ad1f[...])

    mask = (topo != 0).astype(jnp.float32)
    rowsum = jnp.sum(mask, axis=1, keepdims=True)
    nnz = jnp.dot(segf[...], rowsum)
    padc = jax.lax.dot_general(100.0 - nnz, segf[...],
                               (((0,), (0,)), ((), ())))
    maskt = mask.T

    CTs = []
    for t in range(SUB):
        sl = slice(RN * t, RN * (t + 1))
        CTs.append(jnp.dot(rept[...], maskt[:, sl]) * samegf[...]
                   + ind0f[...] * padc[:, sl])

    x1_parts = []
    for t in range(SUB):
        sl = slice(RN * t, RN * (t + 1))
        outs = [_gat_head(hw1[sl, 64 * h:64 * (h + 1)],
                          as1[h:h + 1, :], edall1[sl, h:h + 1], CTs[t])
                for h in range(4)]
        x1_parts.append(jnp.concatenate(outs, axis=1))
    x1 = jnp.concatenate(x1_parts, axis=0) + bg1[...]
    x1 = _elu(_ln(x1, g1[...], b1[...]))

    hw2 = jnp.dot(x1, wg2[...])
    edall2 = jnp.dot(hw2, ad2f[...])
    x2 = jnp.concatenate(
        [_gat_head(hw2[RN * t:RN * (t + 1)], as2[...],
                   edall2[RN * t:RN * (t + 1)], CTs[t]) for t in range(SUB)],
        axis=0) + bg2[...]
    x2 = _ln(x2, g2[...], b2[...])

    outg = _elu(x2 + ident)
    g = jnp.dot(segf[...], outg) * 0.1

    r = _ln(jax.nn.relu(jnp.dot(rt_ref[0], wr[...]) + br[...]),
            gr[...], brb[...])
    tt = _ln(jax.nn.relu(jnp.dot(tf_ref[0], wt[...]) + bt[...]),
             gt[...], btb[...])
    comb = jnp.concatenate([g, r, tt], axis=1)
    feat = _ln(jax.nn.relu(jnp.dot(comb, wf[...]) + bf[...]),
               gf[...], bfb[...])
    h1 = jax.nn.relu(jnp.dot(feat, wa1[...]) + ba1[...])
    h2 = jax.nn.relu(jnp.dot(h1, wa2[...]) + ba2[...])
    out_ref[0] = jnp.dot(h2, wa3[...]) + ba3[...]


def _full(a):
    a = jnp.asarray(a, jnp.float32)
    if a.ndim == 1:
        a = a.reshape(1, -1)
    return pl.BlockSpec(a.shape, lambda i: (0,) * a.ndim), a


@jax.jit
def kernel(x, params):
    p = params
    B = x.shape[0]
    xp = jnp.pad(x, ((0, PB - B), (0, 0)))
    topo = xp[:, :100].reshape(STEPS, ROWS, 10)
    nf = jnp.concatenate(
        [xp[:, 100:130].reshape(PB, 10, 3), xp[:, 245:255].reshape(PB, 10, 1)],
        axis=-1).reshape(STEPS, ROWS, 4)
    routing = xp[:, 130:140].reshape(STEPS, SPG, 10)
    traffic = xp[:, 240:245].reshape(STEPS, SPG, 5)

    s_of_row = np.arange(ROWS) // 10
    segf = (s_of_row[None, :] == np.arange(SPG)[:, None]).astype(np.float32)
    r1 = np.arange(RN)
    samegf = ((r1[:, None] // 10) == (r1[None, :] // 10)).astype(np.float32)
    ind0f = ((r1[:, None] % 10 == 0) & (r1[None, :] == r1[:, None])
             ).astype(np.float32)
    rept = ((r1[:, None] % 10) == np.arange(10)[None, :]).astype(np.float32)

    pnames = ['we', 'be', 'gne', 'bne', 'wp', 'bp',
              'wg1', 'as1', 'ad1f', 'bg1', 'g1', 'b1',
              'wg2', 'as2', 'ad2f', 'bg2', 'g2', 'b2',
              'wr', 'br', 'gr', 'brb', 'wt', 'bt', 'gt', 'btb',
              'wf', 'bf', 'gf', 'bfb', 'wa1', 'ba1', 'wa2', 'ba2',
              'wa3', 'ba3']
    cspecs, cvals = zip(*(_full(a) for a in
                          (segf, samegf, ind0f, rept)))
    px = dict(p)
    px['ad1f'] = (jnp.eye(4)[:, None, :] * p['ad1'][:, :, None]).reshape(256, 4)
    px['ad2f'] = p['ad2'].reshape(64, 1)
    pspecs, pvals = zip(*(_full(px[n]) for n in pnames))

    out = pl.pallas_call(
        _actor_kernel,
        grid=(STEPS,),
        in_specs=[pl.BlockSpec((1, ROWS, 4), lambda i: (i, 0, 0)),
                  pl.BlockSpec((1, ROWS, 10), lambda i: (i, 0, 0)),
                  pl.BlockSpec((1, SPG, 10), lambda i: (i, 0, 0)),
                  pl.BlockSpec((1, SPG, 5), lambda i: (i, 0, 0)),
                  *cspecs, *pspecs],
        out_specs=pl.BlockSpec((1, SPG, 10), lambda i: (i, 0, 0)),
        out_shape=jax.ShapeDtypeStruct((STEPS, SPG, 10), jnp.float32),
    )(nf, topo, routing, traffic, *cvals, *pvals)
    return out.reshape(PB, 10)[:B]
